# baseline (speedup 1.0000x reference)
import functools

import jax
import jax.numpy as jnp
from jax import lax
from jax.experimental import pallas as pl
from jax.experimental.pallas import tpu as pltpu
from jax.experimental.pallas import tpu_sc as plsc

NC = 2
NS = 16
NW = NC * NS

CHUNK = 512


def _sc_gather(idx3, table, total, b_per_w, nchunk, chunk):
    d = table.shape[1]
    mesh = plsc.VectorSubcoreMesh(core_axis_name="c", subcore_axis_name="s")

    @functools.partial(
        pl.kernel,
        mesh=mesh,
        compiler_params=pltpu.CompilerParams(use_tc_tiling_on_sc=False),
        out_type=jax.ShapeDtypeStruct((total, d), jnp.float32),
        scratch_types=[
            pltpu.VMEM((chunk,), jnp.int32),
            pltpu.VMEM((chunk,), jnp.int32),
            pltpu.VMEM((chunk, d), jnp.float32),
            pltpu.VMEM((chunk, d), jnp.float32),
            pltpu.SemaphoreType.DMA,
            pltpu.SemaphoreType.DMA,
        ],
    )
    def gather_kernel(idx_hbm, table_hbm, out_hbm, idx0, idx1, buf0, buf1, sem0, sem1):
        wid = lax.axis_index("s") * NC + lax.axis_index("c")
        base = wid * b_per_w
        out_rows = out_hbm
        idxs = (idx0, idx1)
        bufs = (buf0, buf1)
        sems = (sem0, sem1)

        def start(j, s):
            pltpu.sync_copy(idx_hbm.at[wid, j], idxs[s])
            return pltpu.async_copy(table_hbm.at[idxs[s]], bufs[s], sems[s])

        cps = [start(0, 0), None]
        for j in range(nchunk):
            s = j & 1
            if j + 1 < nchunk:
                cps[(j + 1) & 1] = start(j + 1, (j + 1) & 1)
            cps[s].wait()
            pltpu.sync_copy(bufs[s], out_rows.at[pl.ds(base + j * chunk, chunk)])

    return gather_kernel(idx3, table)


def _tc_linear_t(g128, W, b2, L, B):
    half = B // 2
    bk = 512
    npc = half // bk

    def body(g_ref, w_ref, b_ref, o_ref):
        w = w_ref[...]
        bias = b_ref[...]
        for j in range(npc):
            blk = g_ref[pl.ds(j * bk, bk), :]
            e = lax.dot_general(
                w, blk[:, :64], (((1,), (1,)), ((), ())),
                preferred_element_type=jnp.float32,
            )
            o_ref[0, :, pl.ds(j * bk, bk)] = e + bias
            o = lax.dot_general(
                w, blk[:, 64:], (((1,), (1,)), ((), ())),
                preferred_element_type=jnp.float32,
            )
            o_ref[0, :, pl.ds(half + j * bk, bk)] = o + bias

    return pl.pallas_call(
        body,
        grid=(L,),
        in_specs=[
            pl.BlockSpec((half, 128), lambda i: (i, 0)),
            pl.BlockSpec((64, 64), lambda i: (0, 0)),
            pl.BlockSpec((64, 1), lambda i: (0, 0)),
        ],
        out_specs=pl.BlockSpec((1, 64, B), lambda i: (i, 0, 0)),
        out_shape=jax.ShapeDtypeStruct((L, 64, B), jnp.float32),
    )(g128, W, b2)


def kernel(x, embed_table, W, b):
    bsz, hist = x.shape
    d = embed_table.shape[1]
    total = bsz * hist

    b_per_w = total // NW
    chunk = CHUNK
    nchunk = b_per_w // chunk
    assert b_per_w % chunk == 0 and total % NW == 0 and bsz % 2 == 0

    idx = (
        x.astype(jnp.int32)
        .T.reshape(hist, 2, bsz // 2)
        .transpose(0, 2, 1)
        .reshape(NW, nchunk, chunk)
    )
    gathered = _sc_gather(idx, embed_table, total, b_per_w, nchunk, chunk)
    g128 = gathered.reshape(total // 2, 2 * d)

    out_t = _tc_linear_t(g128, W, b.reshape(d, 1), hist, bsz)
    return jnp.transpose(out_t, (2, 0, 1))


# speedup vs baseline: 1.0486x; 1.0486x over previous
import functools

import jax
import jax.numpy as jnp
from jax import lax
from jax.experimental import pallas as pl
from jax.experimental.pallas import tpu as pltpu
from jax.experimental.pallas import tpu_sc as plsc

NC = 2
NS = 16
NW = NC * NS

CHUNK = 512


def _sc_gather(idx_flat, table, total, bsz, b_per_w, nchunk, chunk):
    d = table.shape[1]
    half = bsz // 2
    mesh = plsc.VectorSubcoreMesh(core_axis_name="c", subcore_axis_name="s")

    @functools.partial(
        pl.kernel,
        mesh=mesh,
        compiler_params=pltpu.CompilerParams(use_tc_tiling_on_sc=False),
        out_type=jax.ShapeDtypeStruct((total // 2, 2 * d), jnp.float32),
        scratch_types=[
            pltpu.VMEM((chunk,), jnp.int32),
            pltpu.VMEM((chunk,), jnp.int32),
            pltpu.VMEM((chunk, d), jnp.float32),
            pltpu.VMEM((chunk, d), jnp.float32),
            pltpu.SemaphoreType.DMA,
            pltpu.SemaphoreType.DMA,
        ],
    )
    def gather_kernel(idx_hbm, table_hbm, out_hbm, idx0, idx1, buf0, buf1, sem0, sem1):
        wid = lax.axis_index("s") * NC + lax.axis_index("c")
        base = wid * b_per_w
        idxs = (idx0, idx1)
        bufs = (buf0, buf1)
        sems = (sem0, sem1)

        def start(j, s):
            pltpu.sync_copy(idx_hbm.at[pl.ds(base + j * chunk, chunk)], idxs[s])
            return pltpu.async_copy(table_hbm.at[idxs[s]], bufs[s], sems[s])

        cps = [start(0, 0), None]
        for j in range(nchunk):
            s = j & 1
            if j + 1 < nchunk:
                cps[(j + 1) & 1] = start(j + 1, (j + 1) & 1)
            cps[s].wait()
            p0 = base + j * chunk
            l = p0 // bsz
            off = p0 - l * bsz
            hi = off // half
            row0 = l * half + off - hi * half
            pltpu.sync_copy(
                bufs[s], out_hbm.at[pl.ds(row0, chunk), pl.ds(hi * d, d)]
            )

    return gather_kernel(idx_flat, table)


def _tc_linear_t(g128, W, b2, L, B):
    half = B // 2
    bk = 512
    npc = half // bk

    def body(g_ref, w_ref, b_ref, o_ref):
        w = w_ref[...]
        bias = b_ref[...]
        for j in range(npc):
            blk = g_ref[pl.ds(j * bk, bk), :]
            e = lax.dot_general(
                w, blk[:, :64], (((1,), (1,)), ((), ())),
                preferred_element_type=jnp.float32,
            )
            o_ref[0, :, pl.ds(j * bk, bk)] = e + bias
            o = lax.dot_general(
                w, blk[:, 64:], (((1,), (1,)), ((), ())),
                preferred_element_type=jnp.float32,
            )
            o_ref[0, :, pl.ds(half + j * bk, bk)] = o + bias

    return pl.pallas_call(
        body,
        grid=(L,),
        in_specs=[
            pl.BlockSpec((half, 128), lambda i: (i, 0)),
            pl.BlockSpec((64, 64), lambda i: (0, 0)),
            pl.BlockSpec((64, 1), lambda i: (0, 0)),
        ],
        out_specs=pl.BlockSpec((1, 64, B), lambda i: (i, 0, 0)),
        out_shape=jax.ShapeDtypeStruct((L, 64, B), jnp.float32),
    )(g128, W, b2)


def kernel(x, embed_table, W, b):
    bsz, hist = x.shape
    d = embed_table.shape[1]
    total = bsz * hist

    b_per_w = total // NW
    chunk = CHUNK
    nchunk = b_per_w // chunk
    assert b_per_w % chunk == 0 and total % NW == 0 and bsz % 2 == 0
    assert (bsz // 2) % chunk == 0 and b_per_w % chunk == 0

    idx_flat = x.astype(jnp.int32).T.reshape(total)
    g128 = _sc_gather(idx_flat, embed_table, total, bsz, b_per_w, nchunk, chunk)

    out_t = _tc_linear_t(g128, W, b.reshape(d, 1), hist, bsz)
    return jnp.transpose(out_t, (2, 0, 1))
